# Initial kernel scaffold; baseline (speedup 1.0000x reference)
import jax
import jax.numpy as jnp
from jax.experimental import pallas as pl


def kernel(inputs, cluster_mean, cluster_size, cluster_sum):
    raise NotImplementedError("write your pallas kernel here")


# speedup vs baseline: 1.5798x; 1.5798x over previous
import functools

import jax
import jax.numpy as jnp
from jax.experimental import pallas as pl
from jax.experimental.pallas import tpu as pltpu

EMBED_DIM = 256
N_EMBED = 8192
DECAY = 0.99
EPS = 1e-05

S = 8 * 576
CODE_BLK = 1024
N_CODE_BLKS = N_EMBED // CODE_BLK


def _argmin_body(samples_ref, mean_ref, csize_ref, idx_ref, sumcs_ref,
                 best_ref, barg_ref, s2_ref):
    j = pl.program_id(0)
    samples = samples_ref[...]

    @pl.when(j == 0)
    def _init():
        s2_ref[...] = jnp.sum(samples * samples, axis=1, keepdims=True)
        best_ref[...] = jnp.full((S, 1), jnp.inf, jnp.float32)
        barg_ref[...] = jnp.zeros((S, 1), jnp.int32)
        sumcs_ref[...] = jnp.sum(csize_ref[...], keepdims=True).reshape(1, 1)

    mean = mean_ref[...]
    mm = jnp.dot(samples, mean, preferred_element_type=jnp.float32)
    m2 = jnp.sum(mean * mean, axis=0, keepdims=True)
    dist = s2_ref[...] - 2.0 * mm + m2

    local_min = jnp.min(dist, axis=1, keepdims=True)
    col = jax.lax.broadcasted_iota(jnp.int32, dist.shape, 1)
    local_arg = jnp.min(
        jnp.where(dist == local_min, col, jnp.int32(2**30)),
        axis=1, keepdims=True) + j * CODE_BLK

    better = local_min < best_ref[...]
    best_ref[...] = jnp.where(better, local_min, best_ref[...])
    barg_ref[...] = jnp.where(better, local_arg, barg_ref[...])

    @pl.when(j == N_CODE_BLKS - 1)
    def _done():
        idx_ref[...] = barg_ref[...]


def _assemble_body(samples_ref, idx_ref, csum_ref, csize_ref, ns_ref,
                   out_ref, loss_ref, q_ref):
    j = pl.program_id(0)
    idx = idx_ref[...]
    col = jax.lax.broadcasted_iota(jnp.int32, (S, CODE_BLK), 1) + j * CODE_BLK
    onehot = (idx == col).astype(jnp.float32)

    counts = jnp.sum(onehot, axis=0, keepdims=True)
    samples = samples_ref[...]
    bsum = jax.lax.dot_general(
        samples, onehot, (((0,), (0,)), ((), ())),
        preferred_element_type=jnp.float32)

    usum = csum_ref[...] * DECAY + bsum * (1.0 - DECAY)
    usize = csize_ref[...] * DECAY + counts * (1.0 - DECAY)
    ns = ns_ref[0, 0]
    smoothed = (usize + EPS) * ns / (ns + N_EMBED * EPS)
    new_mean = usum / smoothed

    part = jax.lax.dot_general(
        onehot, new_mean, (((1,), (1,)), ((), ())),
        preferred_element_type=jnp.float32)

    @pl.when(j == 0)
    def _init():
        q_ref[...] = jnp.zeros((S, EMBED_DIM), jnp.float32)

    q_ref[...] += part

    @pl.when(j == N_CODE_BLKS - 1)
    def _done():
        q = q_ref[...]
        out_ref[...] = q
        diff = samples - q
        loss_ref[...] = jnp.sum(diff * diff, keepdims=True).reshape(1, 1)


@functools.partial(jax.jit, static_argnames=("interpret",))
def kernel(inputs, cluster_mean, cluster_size, cluster_sum, interpret=False):
    samples = jnp.reshape(inputs, (S, EMBED_DIM))
    csize_2d = jnp.reshape(cluster_size, (1, N_EMBED))

    idx, sumcs = pl.pallas_call(
        _argmin_body,
        grid=(N_CODE_BLKS,),
        in_specs=[
            pl.BlockSpec((S, EMBED_DIM), lambda j: (0, 0)),
            pl.BlockSpec((EMBED_DIM, CODE_BLK), lambda j: (0, j)),
            pl.BlockSpec((1, N_EMBED), lambda j: (0, 0)),
        ],
        out_specs=[
            pl.BlockSpec((S, 1), lambda j: (0, 0)),
            pl.BlockSpec((1, 1), lambda j: (0, 0)),
        ],
        out_shape=[
            jax.ShapeDtypeStruct((S, 1), jnp.int32),
            jax.ShapeDtypeStruct((1, 1), jnp.float32),
        ],
        scratch_shapes=[
            pltpu.VMEM((S, 1), jnp.float32),
            pltpu.VMEM((S, 1), jnp.int32),
            pltpu.VMEM((S, 1), jnp.float32),
        ],
        interpret=interpret,
    )(samples, cluster_mean, csize_2d)

    n_sample = sumcs * DECAY + (1.0 - DECAY) * float(S)

    out, loss = pl.pallas_call(
        _assemble_body,
        grid=(N_CODE_BLKS,),
        in_specs=[
            pl.BlockSpec((S, EMBED_DIM), lambda j: (0, 0)),
            pl.BlockSpec((S, 1), lambda j: (0, 0)),
            pl.BlockSpec((EMBED_DIM, CODE_BLK), lambda j: (0, j)),
            pl.BlockSpec((1, CODE_BLK), lambda j: (0, j)),
            pl.BlockSpec((1, 1), lambda j: (0, 0)),
        ],
        out_specs=[
            pl.BlockSpec((S, EMBED_DIM), lambda j: (0, 0)),
            pl.BlockSpec((1, 1), lambda j: (0, 0)),
        ],
        out_shape=[
            jax.ShapeDtypeStruct((S, EMBED_DIM), jnp.float32),
            jax.ShapeDtypeStruct((1, 1), jnp.float32),
        ],
        scratch_shapes=[
            pltpu.VMEM((S, EMBED_DIM), jnp.float32),
        ],
        interpret=interpret,
    )(samples, idx, cluster_sum, csize_2d, n_sample)

    outputs = jnp.reshape(out, inputs.shape)
    e_loss = loss[0, 0] / float(S * EMBED_DIM)
    return (outputs, 0.25 * e_loss)
